# baseline (speedup 1.0000x reference)
import functools

import jax
import jax.numpy as jnp
from jax import lax
from jax.experimental import pallas as pl
from jax.experimental.pallas import tpu as pltpu
from jax.experimental.pallas import tpu_sc as plsc

BATCH = 16384
HIDDEN = 64
NUM_NODES = 1000000
L = 16

_info = plsc.get_sparse_core_info()
NC, NS = _info.num_cores, _info.num_subcores
NW = NC * NS

BLK = 128
NFULL = NUM_NODES // BLK
NTAIL = NUM_NODES - NFULL * BLK
BPT = (NFULL + NW - 1) // NW

BPW = BATCH // NW
CH = 32
NCHUNK = BPW // CH
NBUF = 4

_mesh = plsc.VectorSubcoreMesh(core_axis_name="c", subcore_axis_name="s")


@functools.partial(
    pl.kernel,
    mesh=_mesh,
    out_type=jax.ShapeDtypeStruct((NUM_NODES, HIDDEN), jnp.float32),
    compiler_params=pltpu.CompilerParams(needs_layout_passes=False),
    scratch_types=[
        pltpu.VMEM((HIDDEN, BLK), jnp.float32),
        pltpu.VMEM((HIDDEN, BLK), jnp.float32),
        pltpu.VMEM((BLK, HIDDEN), jnp.float32),
        pltpu.VMEM((BLK, HIDDEN), jnp.float32),
        pltpu.SemaphoreType.DMA,
        pltpu.SemaphoreType.DMA,
    ],
)
def _transpose_table(nodet_hbm, rowtab_hbm, blk0, blk1, out0, out1,
                     sem_f, sem_w):
    wid = lax.axis_index("s") * NC + lax.axis_index("c")
    lo = wid * BPT
    n = jnp.minimum(BPT, NFULL - lo)

    lanes = lax.iota(jnp.int32, L)
    jlanes = [j0 + lanes for j0 in range(0, HIDDEN, L)]

    def fetch(b, blk):
        col = pl.multiple_of((lo + b) * BLK, BLK)
        pltpu.async_copy(nodet_hbm.at[:, pl.ds(col, BLK)], blk, sem_f)

    def wait_fetch(blk):
        pltpu.make_async_copy(nodet_hbm.at[:, pl.ds(0, BLK)], blk,
                              sem_f).wait()

    def transpose_rows(blk, out, nrows):
        def row_body(r, carry):
            rv = jnp.full((L,), 0, jnp.int32) + r
            for g, jl in enumerate(jlanes):
                out[r, pl.ds(g * L, L)] = plsc.load_gather(blk, [jl, rv])
            return carry

        lax.fori_loop(0, nrows, row_body, 0, unroll=4)

    def writeback(b, out, nrows):
        row0 = pl.multiple_of((lo + b) * BLK, BLK)
        pltpu.async_copy(out.at[pl.ds(0, nrows)],
                         rowtab_hbm.at[pl.ds(row0, nrows)], sem_w)

    def drain_write(out, nrows):
        pltpu.make_async_copy(out.at[pl.ds(0, nrows)],
                              rowtab_hbm.at[pl.ds(0, nrows)], sem_w).wait()

    @pl.when(n > 0)
    def _():
        fetch(0, blk0)

    def phase(b, blk, out):
        @pl.when(b < n)
        def _():
            @pl.when(b + 1 < n)
            def _():
                fetch(b + 1, blk1 if blk is blk0 else blk0)

            wait_fetch(blk)

            @pl.when(b >= 2)
            def _():
                drain_write(out, BLK)

            transpose_rows(blk, out, BLK)
            writeback(b, out, BLK)

    def loop_body(i, carry):
        phase(2 * i, blk0, out0)
        phase(2 * i + 1, blk1, out1)
        return carry

    lax.fori_loop(0, (BPT + 1) // 2, loop_body, 0)

    @pl.when(n >= 2)
    def _():
        drain_write(out0, BLK)
        drain_write(out1, BLK)


@functools.partial(
    pl.kernel,
    mesh=_mesh,
    out_type=jax.ShapeDtypeStruct((BATCH,), jnp.float32),
    compiler_params=pltpu.CompilerParams(needs_layout_passes=False),
    scratch_types=[
        pltpu.VMEM((3 * BPW,), jnp.int32),
        pltpu.VMEM((BPW,), jnp.float32),
        pltpu.VMEM((NBUF, CH, HIDDEN), jnp.float32),
        pltpu.VMEM((NBUF, CH, HIDDEN), jnp.float32),
        pltpu.VMEM((NBUF, CH, HIDDEN), jnp.float32),
        pltpu.SemaphoreType.DMA((NBUF,)),
    ],
)
def _kge_score_sc(head_hbm, rel_hbm, tail_hbm, node_hbm, relemb_hbm,
                  tail_hbm_rows, out_hbm,
                  idx3, scores, hbufs, rbufs, tbufs, sems):
    sid = lax.axis_index("s")
    wid = sid * NC + lax.axis_index("c")
    base = wid * BPW

    @pl.when(sid == 0)
    def _():
        for half in range(NTAIL // CH):
            stage = hbufs.at[0]
            pltpu.sync_copy(tail_hbm_rows.at[pl.ds(half * CH, CH)], stage)
            pltpu.sync_copy(
                stage, node_hbm.at[pl.ds(NFULL * BLK + half * CH, CH)])

    plsc.subcore_barrier()

    pltpu.sync_copy(head_hbm.at[pl.ds(base, BPW)], idx3.at[pl.ds(0, BPW)])
    pltpu.sync_copy(rel_hbm.at[pl.ds(base, BPW)], idx3.at[pl.ds(BPW, BPW)])
    pltpu.sync_copy(tail_hbm.at[pl.ds(base, BPW)],
                    idx3.at[pl.ds(2 * BPW, BPW)])

    lanes = lax.iota(jnp.int32, L)

    def fire(chunk, b):
        off = pl.multiple_of(chunk * CH, CH)
        for g in range(CH // L):
            ihv = idx3[pl.ds(off + g * L, L)]
            irv = idx3[pl.ds(BPW + off + g * L, L)]
            itv = idx3[pl.ds(2 * BPW + off + g * L, L)]
            for k in range(L):
                dst = pl.ds(g * L + k, 1)
                pltpu.async_copy(node_hbm.at[pl.ds(ihv[k], 1), :],
                                 hbufs.at[b, dst, :], sems.at[b])
                pltpu.async_copy(relemb_hbm.at[pl.ds(irv[k], 1), :],
                                 rbufs.at[b, dst, :], sems.at[b])
                pltpu.async_copy(node_hbm.at[pl.ds(itv[k], 1), :],
                                 tbufs.at[b, dst, :], sems.at[b])

    def drain_and_compute(chunk, b):
        for bufs in (hbufs, rbufs, tbufs):
            pltpu.make_async_copy(node_hbm.at[pl.ds(0, CH), :], bufs.at[b],
                                  sems.at[b]).wait()
        bsel = jnp.full((L,), b, dtype=jnp.int32)

        for g in range(CH // L):
            rows = g * L + lanes

            def col_body(j, acc, rows=rows):
                cj = jnp.full((L,), j, dtype=jnp.int32)
                h = plsc.load_gather(hbufs, [bsel, rows, cj])
                r = plsc.load_gather(rbufs, [bsel, rows, cj])
                t = plsc.load_gather(tbufs, [bsel, rows, cj])
                return acc + jnp.abs(h + r - t)

            acc = lax.fori_loop(0, HIDDEN, col_body,
                                jnp.zeros((L,), jnp.float32), unroll=2)
            scores[pl.ds(pl.multiple_of(chunk * CH + g * L, L), L)] = -acc

    for b in range(NBUF - 1):
        fire(b, b)

    def ring_body(c, carry):
        for b in range(NBUF):
            nxt = c + b + (NBUF - 1)

            @pl.when(nxt < NCHUNK)
            def _():
                fire(nxt, (b + NBUF - 1) % NBUF)

            drain_and_compute(c + b, b)
        return carry

    lax.fori_loop(0, NCHUNK // NBUF, lambda i, cy: ring_body(i * NBUF, cy), 0)

    pltpu.sync_copy(scores, out_hbm.at[pl.ds(base, BPW)])


def kernel(head_index, rel_type, tail_index, node_emb, rel_emb):
    rowtab = _transpose_table(node_emb.T)
    return _kge_score_sc(
        head_index.astype(jnp.int32),
        rel_type.astype(jnp.int32),
        tail_index.astype(jnp.int32),
        rowtab,
        rel_emb,
        node_emb[NFULL * BLK:],
    )


# speedup vs baseline: 3.8424x; 3.8424x over previous
import functools

import jax
import jax.numpy as jnp
from jax import lax
from jax.experimental import pallas as pl
from jax.experimental.pallas import tpu as pltpu
from jax.experimental.pallas import tpu_sc as plsc

BATCH = 16384
HIDDEN = 64
L = 16

_info = plsc.get_sparse_core_info()
NC, NS = _info.num_cores, _info.num_subcores
NW = NC * NS
BPW = BATCH // NW
CH = 32
NCHUNK = BPW // CH
NBUF = 4

_mesh = plsc.VectorSubcoreMesh(core_axis_name="c", subcore_axis_name="s")

_row_bufs = [pltpu.VMEM((CH, HIDDEN), jnp.float32)
             for _ in range(3 * NBUF)]


@functools.partial(
    pl.kernel,
    mesh=_mesh,
    out_type=jax.ShapeDtypeStruct((BATCH,), jnp.float32),
    compiler_params=pltpu.CompilerParams(needs_layout_passes=False),
    scratch_types=[
        pltpu.VMEM((BPW,), jnp.int32),
        pltpu.VMEM((BPW,), jnp.int32),
        pltpu.VMEM((BPW,), jnp.int32),
        pltpu.VMEM((BPW,), jnp.float32),
    ] + _row_bufs + [pltpu.SemaphoreType.DMA for _ in range(NBUF)],
)
def _kge_score_sc(head_hbm, rel_hbm, tail_hbm, node_hbm, relemb_hbm, out_hbm,
                  idx_h, idx_r, idx_t, scores, *bufs_and_sems):
    bufs = [bufs_and_sems[3 * b:3 * b + 3] for b in range(NBUF)]
    sems = bufs_and_sems[3 * NBUF:]

    wid = lax.axis_index("s") * NC + lax.axis_index("c")
    base = wid * BPW

    pltpu.sync_copy(head_hbm.at[pl.ds(base, BPW)], idx_h)
    pltpu.sync_copy(rel_hbm.at[pl.ds(base, BPW)], idx_r)
    pltpu.sync_copy(tail_hbm.at[pl.ds(base, BPW)], idx_t)

    lanes = lax.iota(jnp.int32, L)

    def fire(chunk, b):
        off = pl.multiple_of(chunk * CH, CH)
        hbuf, rbuf, tbuf = bufs[b]
        for g in range(CH // L):
            ihv = idx_h[pl.ds(off + g * L, L)]
            irv = idx_r[pl.ds(off + g * L, L)]
            itv = idx_t[pl.ds(off + g * L, L)]
            for k in range(L):
                dst = pl.ds(g * L + k, 1)
                pltpu.async_copy(node_hbm.at[pl.ds(ihv[k], 1), :],
                                 hbuf.at[dst], sems[b])
                pltpu.async_copy(relemb_hbm.at[pl.ds(irv[k], 1), :],
                                 rbuf.at[dst], sems[b])
                pltpu.async_copy(node_hbm.at[pl.ds(itv[k], 1), :],
                                 tbuf.at[dst], sems[b])

    def drain_and_compute(chunk, b):
        hbuf, rbuf, tbuf = bufs[b]
        for buf in (hbuf, rbuf, tbuf):
            pltpu.make_async_copy(node_hbm.at[pl.ds(0, CH), :], buf,
                                  sems[b]).wait()

        for g in range(CH // L):
            rows = g * L + lanes

            def col_body(j, acc, rows=rows):
                cj = jnp.full((L,), j, dtype=jnp.int32)
                h = plsc.load_gather(hbuf, [rows, cj])
                r = plsc.load_gather(rbuf, [rows, cj])
                t = plsc.load_gather(tbuf, [rows, cj])
                return acc + jnp.abs(h + r - t)

            acc = lax.fori_loop(0, HIDDEN, col_body,
                                jnp.zeros((L,), jnp.float32), unroll=2)
            scores[pl.ds(pl.multiple_of(chunk * CH + g * L, L), L)] = -acc

    for b in range(NBUF - 1):
        fire(b, b)

    def ring_body(c, carry):
        for b in range(NBUF):
            nxt = c + b + (NBUF - 1)

            @pl.when(nxt < NCHUNK)
            def _():
                fire(nxt, (b + NBUF - 1) % NBUF)

            drain_and_compute(c + b, b)
        return carry

    lax.fori_loop(0, NCHUNK // NBUF, lambda i, cy: ring_body(i * NBUF, cy), 0)

    pltpu.sync_copy(scores, out_hbm.at[pl.ds(base, BPW)])


def kernel(head_index, rel_type, tail_index, node_emb, rel_emb):
    return _kge_score_sc(
        head_index.astype(jnp.int32),
        rel_type.astype(jnp.int32),
        tail_index.astype(jnp.int32),
        node_emb,
        rel_emb,
    )
